# Initial kernel scaffold; baseline (speedup 1.0000x reference)
import jax
import jax.numpy as jnp
from jax.experimental import pallas as pl


def kernel(z, emb_weight):
    raise NotImplementedError("write your pallas kernel here")


# speedup vs baseline: 1.2052x; 1.2052x over previous
import functools

import jax
import jax.numpy as jnp
from jax import lax
from jax.experimental import pallas as pl
from jax.experimental.pallas import tpu as pltpu
from jax.experimental.pallas import tpu_sc as plsc

N_E = 8192
E_DIM = 64
N_TOK = 16384
HALF = N_E // 2

TOK_TILE = 256
N_TILES = N_TOK // TOK_TILE


def _half_min(zb, eb_half, zn_en_half, idx_base):
    mm = lax.dot_general(zb, eb_half, (((1,), (1,)), ((), ())),
                         preferred_element_type=jnp.float32)
    d = zn_en_half - 2.0 * mm
    mv = jnp.min(d, axis=1, keepdims=True)
    ids = lax.broadcasted_iota(jnp.int32, d.shape, 1) + idx_base
    mi = jnp.min(jnp.where(d == mv, ids, jnp.int32(N_E)), axis=1)
    return mv[:, 0], mi


def _argmin_body(zb_ref, eb_ref, zn_ref, en_ref, o_ref):
    zb = zb_ref[...]
    eb = eb_ref[...]
    zn = zn_ref[...].reshape(TOK_TILE, 1)
    en = en_ref[...]
    t = zn + en[None, :]
    v0, i0 = _half_min(zb, eb[:HALF], t[:, :HALF], 0)
    v1, i1 = _half_min(zb, eb[HALF:], t[:, HALF:], HALF)
    v0c = v0.astype(jnp.bfloat16).astype(jnp.float32)
    idx = jnp.where(v0c <= v1, i0, i1)
    o_ref[...] = idx.reshape(1, 1, TOK_TILE)


def _tc_argmin(zb, eb, zn, en):
    out = pl.pallas_call(
        _argmin_body,
        grid=(N_TILES,),
        in_specs=[
            pl.BlockSpec((TOK_TILE, E_DIM), lambda i: (i, 0)),
            pl.BlockSpec((N_E, E_DIM), lambda i: (0, 0)),
            pl.BlockSpec((TOK_TILE,), lambda i: (i,)),
            pl.BlockSpec((N_E,), lambda i: (0,)),
        ],
        out_specs=pl.BlockSpec((1, 1, TOK_TILE), lambda i: (i, 0, 0)),
        out_shape=jax.ShapeDtypeStruct((N_TILES, 1, TOK_TILE), jnp.int32),
    )(zb, eb, zn, en)
    return out.reshape(N_TOK)


def _make_sc_gather():
    info = plsc.get_sparse_core_info()
    nw = info.num_cores * info.num_subcores
    b_per_w = N_TOK // nw
    chunk = 128
    n_chunks = b_per_w // chunk
    mesh = plsc.VectorSubcoreMesh(core_axis_name="c", subcore_axis_name="s")

    @functools.partial(
        pl.kernel,
        mesh=mesh,
        out_type=jax.ShapeDtypeStruct((N_TOK, E_DIM), jnp.float32),
        scratch_types=[
            pltpu.VMEM((b_per_w,), jnp.int32),
            pltpu.VMEM((b_per_w, E_DIM), jnp.float32),
            pltpu.SemaphoreType.DMA,
        ],
        compiler_params=pltpu.CompilerParams(use_tc_tiling_on_sc=False),
    )
    def gather(table_hbm, idx_hbm, out_hbm, idx_v, rows_v, sem):
        wid = lax.axis_index("s") * info.num_cores + lax.axis_index("c")
        base = wid * b_per_w
        pltpu.sync_copy(idx_hbm.at[pl.ds(base, b_per_w)], idx_v)
        copies = []
        for j in range(n_chunks):
            copies.append(pltpu.async_copy(
                table_hbm.at[idx_v.at[pl.ds(j * chunk, chunk)]],
                rows_v.at[pl.ds(j * chunk, chunk)], sem))
        for c in copies:
            c.wait()
        pltpu.sync_copy(rows_v, out_hbm.at[pl.ds(base, b_per_w)])

    return gather


def kernel(z, emb_weight):
    zn = jnp.sum(z ** 2, axis=1)
    en = jnp.sum(emb_weight ** 2, axis=1)
    zb = z.astype(jnp.bfloat16)
    eb = emb_weight.astype(jnp.bfloat16)
    idx = _tc_argmin(zb, eb, zn, en)
    zq = _make_sc_gather()(emb_weight, idx)
    z_q = z + (zq - z)
    return (z_q, idx, jnp.zeros((z.shape[0],), dtype=z.dtype))


# speedup vs baseline: 1.2590x; 1.0446x over previous
import functools

import jax
import jax.numpy as jnp
from jax import lax
from jax.experimental import pallas as pl
from jax.experimental.pallas import tpu as pltpu
from jax.experimental.pallas import tpu_sc as plsc

N_E = 8192
E_DIM = 64
N_TOK = 16384
HALF = N_E // 2

TOK_TILE = 256
N_TILES = N_TOK // TOK_TILE


def _half_min(zb2, eb_half, zn_en_half, ids_half):
    mm2 = lax.dot_general(zb2, eb_half, (((1,), (1,)), ((), ())),
                          preferred_element_type=jnp.float32)
    d = zn_en_half - mm2
    mv = jnp.min(d, axis=1, keepdims=True)
    mi = jnp.min(jnp.where(d == mv, ids_half[None, :], jnp.float32(N_E)), axis=1)
    return mv[:, 0], mi


def _argmin_body(zb2_ref, eb_ref, zn_ref, en_ref, ids_ref, o_ref):
    zb2 = zb2_ref[...]
    eb = eb_ref[...]
    zn = zn_ref[...].reshape(TOK_TILE, 1)
    en = en_ref[...]
    ids = ids_ref[...]
    t = zn + en[None, :]
    v0, i0 = _half_min(zb2, eb[:HALF], t[:, :HALF], ids[:HALF])
    v1, i1 = _half_min(zb2, eb[HALF:], t[:, HALF:], ids[HALF:])
    v0c = v0.astype(jnp.bfloat16).astype(jnp.float32)
    idx = jnp.where(v0c <= v1, i0, i1).astype(jnp.int32)
    o_ref[...] = idx.reshape(1, 1, TOK_TILE)


def _tc_argmin(zb2, eb, zn, en, ids):
    out = pl.pallas_call(
        _argmin_body,
        grid=(N_TILES,),
        in_specs=[
            pl.BlockSpec((TOK_TILE, E_DIM), lambda i: (i, 0)),
            pl.BlockSpec((N_E, E_DIM), lambda i: (0, 0)),
            pl.BlockSpec((TOK_TILE,), lambda i: (i,)),
            pl.BlockSpec((N_E,), lambda i: (0,)),
            pl.BlockSpec((N_E,), lambda i: (0,)),
        ],
        out_specs=pl.BlockSpec((1, 1, TOK_TILE), lambda i: (i, 0, 0)),
        out_shape=jax.ShapeDtypeStruct((N_TILES, 1, TOK_TILE), jnp.int32),
    )(zb2, eb, zn, en, ids)
    return out.reshape(N_TOK)


def _make_sc_gather():
    info = plsc.get_sparse_core_info()
    nw = info.num_cores * info.num_subcores
    b_per_w = N_TOK // nw
    chunk = 128
    n_chunks = b_per_w // chunk
    mesh = plsc.VectorSubcoreMesh(core_axis_name="c", subcore_axis_name="s")

    @functools.partial(
        pl.kernel,
        mesh=mesh,
        out_type=jax.ShapeDtypeStruct((N_TOK, E_DIM), jnp.float32),
        scratch_types=[
            pltpu.VMEM((b_per_w,), jnp.int32),
            pltpu.VMEM((b_per_w, E_DIM), jnp.float32),
            pltpu.SemaphoreType.DMA,
        ],
        compiler_params=pltpu.CompilerParams(use_tc_tiling_on_sc=False),
    )
    def gather(table_hbm, idx_hbm, out_hbm, idx_v, rows_v, sem):
        wid = lax.axis_index("s") * info.num_cores + lax.axis_index("c")
        base = wid * b_per_w
        pltpu.sync_copy(idx_hbm.at[pl.ds(base, b_per_w)], idx_v)
        copies = []
        for j in range(n_chunks):
            copies.append(pltpu.async_copy(
                table_hbm.at[idx_v.at[pl.ds(j * chunk, chunk)]],
                rows_v.at[pl.ds(j * chunk, chunk)], sem))
        for c in copies:
            c.wait()
        pltpu.sync_copy(rows_v, out_hbm.at[pl.ds(base, b_per_w)])

    return gather


def kernel(z, emb_weight):
    zn = jnp.sum(z ** 2, axis=1)
    en = jnp.sum(emb_weight ** 2, axis=1)
    zb2 = (2.0 * z).astype(jnp.bfloat16)
    eb = emb_weight.astype(jnp.bfloat16)
    ids = lax.iota(jnp.float32, N_E)
    idx = _tc_argmin(zb2, eb, zn, en, ids)
    zq = _make_sc_gather()(emb_weight, idx)
    z_q = z + (zq - z)
    return (z_q, idx, jnp.zeros((z.shape[0],), dtype=z.dtype))
